# baseline (speedup 1.0000x reference)
import jax
import jax.numpy as jnp
from jax.experimental import pallas as pl

K_SEL = 100


def _sigmoid_body(x_ref, o_ref):
    o_ref[...] = jax.nn.sigmoid(x_ref[...])


def kernel(pred_logits, pred_boxes, target_sizes):
    B, N, C = pred_logits.shape
    flat = pred_logits.reshape(B * N * C)
    blk = 1 << 20
    prob = pl.pallas_call(
        _sigmoid_body,
        grid=((flat.size + blk - 1) // blk,),
        in_specs=[pl.BlockSpec((blk,), lambda i: (i,))],
        out_specs=pl.BlockSpec((blk,), lambda i: (i,)),
        out_shape=jax.ShapeDtypeStruct((flat.size,), jnp.float32),
    )(flat).reshape(B, N * C)
    scores, topk_indexes = jax.lax.top_k(prob, K_SEL)
    topk_boxes = topk_indexes // C
    labels = topk_indexes % C
    b = pred_boxes
    cx, cy, w, h = b[..., 0], b[..., 1], b[..., 2], b[..., 3]
    boxes = jnp.stack([cx - 0.5 * w, cy - 0.5 * h, cx + 0.5 * w, cy + 0.5 * h], axis=-1)
    idx = jnp.broadcast_to(topk_boxes[:, :, None], (B, K_SEL, 4))
    boxes = jnp.take_along_axis(boxes, idx, axis=1)
    img_h = target_sizes[:, 0].astype(jnp.float32)
    img_w = target_sizes[:, 1].astype(jnp.float32)
    scale_fct = jnp.stack([img_w, img_h, img_w, img_h], axis=1)
    boxes = boxes * scale_fct[:, None, :]
    return scores, labels, boxes


# speedup vs baseline: 4.1560x; 4.1560x over previous
import jax
import jax.numpy as jnp
from jax import lax
from jax.experimental import pallas as pl
from jax.experimental.pallas import tpu as pltpu
from jax.experimental.pallas import tpu_sc as plsc

B = 8
N = 20000
C = 91
IMG = N * C
K = 100
SHIFT = 21
NBINS = 2048
UCH = 113744
WLEN = [1424, 1424, 1424, 1424, 1413]
WOFF = [0, 1424, 2848, 4272, 5696]
WBUF = 1424 * 16
REM_OFF = 16 * UCH
REM_V = 6
CAP_T = 512
CAP_M = 1024
LSTRIDE = 2049
NEG = -3.0e38


def _i16():
    return lax.iota(jnp.int32, 16)


def _sortkey(x):
    b = lax.bitcast_convert_type(x, jnp.int32)
    u = lax.shift_right_logical(lax.shift_right_arithmetic(b, 31), 1)
    return b ^ u


def _splat_f(s):
    return jnp.zeros((16,), jnp.float32) + s


def _splat_i(s):
    return jnp.zeros((16,), jnp.int32) + s


def _scal(v):
    return jnp.max(v)


def _body(lg, bx, sc, out_s, out_l, out_b,
          win0, win1, lhist, rhist, hbuf, thbuf, candv, candi, cntrow,
          stagev, stagei, stagec, mergev, mergei, pm, wv, wi, bidx, gbox,
          tmeta, outs_row, outl_row, outb_row, scalev,
          sh_hist, sh_candv, sh_candi, sh_cnt, sh_meta,
          sem0, sem1, semg):
    cid = lax.axis_index("c")
    sid = lax.axis_index("s")
    L = _i16()
    ones = jnp.ones((16,), jnp.int32)
    laneoff = L * LSTRIDE
    is15 = sid == 15
    wins = [win0, win1]
    sems = [sem0, sem1]

    def _image(it, _carry):
        img = cid * 4 + it
        img_off = img * IMG
        base = img_off + sid * UCH
        is_merge = sid == it
        pp = (it & 1) * 16 * CAP_T
        pc_off = (it & 1) * 256

        def _zl(i, _):
            for u in range(8):
                lhist[pl.ds((i * 8 + u) * 16, 16)] = jnp.zeros((16,), jnp.int32)
            return 0
        lax.fori_loop(0, 256, _zl, 0)
        lhist[pl.ds(2048 * 16, 16)] = jnp.zeros((16,), jnp.int32)

        descs = [pltpu.async_copy(lg.at[pl.ds(base, WLEN[0] * 16)],
                                  win0.at[pl.ds(0, WLEN[0] * 16)], sem0)]
        for w in range(5):
            descs[w].wait()
            if w + 1 < 5:
                wb = wins[(w + 1) % 2]
                nlen = WLEN[w + 1] * 16
                descs.append(pltpu.async_copy(
                    lg.at[pl.ds(base + WOFF[w + 1] * 16, nlen)],
                    wb.at[pl.ds(0, nlen)], sems[(w + 1) % 2]))
            wb = wins[w % 2]
            nmain = WLEN[w] // 8

            def _hist8(i, _):
                for u in range(8):
                    x = wb[pl.ds((i * 8 + u) * 16, 16)]
                    bkt = lax.shift_right_arithmetic(_sortkey(x), SHIFT) + 1024
                    plsc.addupdate_scatter(lhist, [laneoff + bkt], ones)
                return 0
            lax.fori_loop(0, nmain, _hist8, 0)

            def _hist1(v, _):
                x = wb[pl.ds(v * 16, 16)]
                bkt = lax.shift_right_arithmetic(_sortkey(x), SHIFT) + 1024
                plsc.addupdate_scatter(lhist, [laneoff + bkt], ones)
                return 0
            lax.fori_loop(nmain * 8, WLEN[w], _hist1, 0)

        pltpu.sync_copy(lg.at[pl.ds(img_off + REM_OFF, REM_V * 16)],
                        win0.at[pl.ds(0, REM_V * 16)])
        m15 = jnp.where(is15, ones > 0, ones < 0)
        for v in range(REM_V):
            x = win0[pl.ds(v * 16, 16)]
            bkt = lax.shift_right_arithmetic(_sortkey(x), SHIFT) + 1024
            plsc.addupdate_scatter(lhist, [laneoff + bkt], ones, mask=m15)

        def _red(r, _):
            acc = lhist[pl.ds(r * 16, 16)]
            for l in range(1, 16):
                acc = acc + lhist[pl.ds(l * LSTRIDE + r * 16, 16)]
            rhist[pl.ds(r * 16, 16)] = acc
            return 0
        lax.fori_loop(0, 128, _red, 0)
        pltpu.sync_copy(rhist, sh_hist.at[pl.ds(sid * NBINS, NBINS)])
        plsc.subcore_barrier()

        @pl.when(is_merge)
        def _():
            pltpu.sync_copy(sh_hist.at[pl.ds(0, NBINS)], thbuf)
            for t in range(1, 16):
                pltpu.sync_copy(sh_hist.at[pl.ds(t * NBINS, NBINS)], hbuf)

                def _acc(r, _):
                    for u in range(8):
                        o = (r * 8 + u) * 16
                        thbuf[pl.ds(o, 16)] = (
                            thbuf[pl.ds(o, 16)] + hbuf[pl.ds(o, 16)])
                    return 0
                lax.fori_loop(0, 16, _acc, 0)

            def _scan(rp, carry):
                total, bstar, done = carry
                r = 127 - rp
                v = thbuf[pl.ds(r * 16, 16)]
                sfx = lax.rev(jnp.cumsum(lax.rev(v, (0,))), (0,))
                mk = (sfx + total) >= K
                npos = _scal(plsc.all_reduce_population_count(mk))
                bstar = jnp.where((done == 0) & (npos > 0),
                                  r * 16 + npos - 1, bstar)
                done = done | jnp.where(npos > 0, 1, 0)
                total = total + jnp.sum(v)
                return total, bstar, done
            _, bstar, _ = lax.fori_loop(0, 128, _scan, (0, 0, 0))
            t_s = lax.shift_left(bstar - 1024, SHIFT)
            tmeta[...] = _splat_i(t_s)
            pltpu.sync_copy(tmeta, sh_meta)
        plsc.subcore_barrier()

        pltpu.sync_copy(sh_meta, tmeta)
        thr = _scal(tmeta[...])

        descs2 = [pltpu.async_copy(lg.at[pl.ds(base, WLEN[0] * 16)],
                                   win0.at[pl.ds(0, WLEN[0] * 16)], sem0)]
        cnt = 0
        for w in range(5):
            descs2[w].wait()
            if w + 1 < 5:
                wb = wins[(w + 1) % 2]
                nlen = WLEN[w + 1] * 16
                descs2.append(pltpu.async_copy(
                    lg.at[pl.ds(base + WOFF[w + 1] * 16, nlen)],
                    wb.at[pl.ds(0, nlen)], sems[(w + 1) % 2]))
            wb = wins[w % 2]
            gbase = sid * UCH + WOFF[w] * 16
            nmain = WLEN[w] // 8

            def _coll8(i, c):
                for u in range(8):
                    v = i * 8 + u
                    x = wb[pl.ds(v * 16, 16)]
                    mk = _sortkey(x) >= thr
                    gi = gbase + v * 16 + L
                    plsc.store_compressed(candv.at[pl.ds(c, 16)], x, mask=mk)
                    plsc.store_compressed(candi.at[pl.ds(c, 16)], gi, mask=mk)
                    npos = _scal(plsc.all_reduce_population_count(mk))
                    c = jnp.minimum(c + npos, CAP_T)
                return c
            cnt = lax.fori_loop(0, nmain, _coll8, cnt)

            def _coll1(v, c):
                x = wb[pl.ds(v * 16, 16)]
                mk = _sortkey(x) >= thr
                gi = gbase + v * 16 + L
                plsc.store_compressed(candv.at[pl.ds(c, 16)], x, mask=mk)
                plsc.store_compressed(candi.at[pl.ds(c, 16)], gi, mask=mk)
                npos = _scal(plsc.all_reduce_population_count(mk))
                return jnp.minimum(c + npos, CAP_T)
            cnt = lax.fori_loop(nmain * 8, WLEN[w], _coll1, cnt)

        pltpu.sync_copy(lg.at[pl.ds(img_off + REM_OFF, REM_V * 16)],
                        win0.at[pl.ds(0, REM_V * 16)])
        for v in range(REM_V):
            x = win0[pl.ds(v * 16, 16)]
            mk = (_sortkey(x) >= thr) & m15
            gi = REM_OFF + v * 16 + L
            plsc.store_compressed(candv.at[pl.ds(cnt, 16)], x, mask=mk)
            plsc.store_compressed(candi.at[pl.ds(cnt, 16)], gi, mask=mk)
            npos = _scal(plsc.all_reduce_population_count(mk))
            cnt = jnp.minimum(cnt + npos, CAP_T)

        cntrow[...] = _splat_i(cnt)
        pltpu.sync_copy(candv.at[pl.ds(0, CAP_T)],
                        sh_candv.at[pl.ds(pp + sid * CAP_T, CAP_T)])
        pltpu.sync_copy(candi.at[pl.ds(0, CAP_T)],
                        sh_candi.at[pl.ds(pp + sid * CAP_T, CAP_T)])
        pltpu.sync_copy(cntrow, sh_cnt.at[pl.ds(pc_off + sid * 16, 16)])
        plsc.subcore_barrier()

        @pl.when(is_merge)
        def _():
            pltpu.sync_copy(sh_candv.at[pl.ds(pp, 16 * CAP_T)], stagev)
            pltpu.sync_copy(sh_candi.at[pl.ds(pp, 16 * CAP_T)], stagei)
            pltpu.sync_copy(sh_cnt.at[pl.ds(pc_off, 256)], stagec)

            def _pre(i, _):
                mergev[pl.ds(i * 16, 16)] = _splat_f(NEG)
                return 0
            lax.fori_loop(0, 66, _pre, 0)

            def _prew(i, _):
                wv[pl.ds(i * 16, 16)] = _splat_f(-20.0)
                wi[pl.ds(i * 16, 16)] = jnp.zeros((16,), jnp.int32)
                bidx[pl.ds(i * 16, 16)] = jnp.zeros((16,), jnp.int32)
                return 0
            lax.fori_loop(0, 8, _prew, 0)

            ptr = 0
            for t in range(16):
                nt = _scal(stagec[pl.ds(t * 16, 16)])
                nvv = lax.shift_right_logical(nt + 15, 4)

                def _ck(k, p):
                    v = stagev[pl.ds(t * CAP_T + k * 16, 16)]
                    i = stagei[pl.ds(t * CAP_T + k * 16, 16)]
                    mk = L < (nt - k * 16)
                    plsc.store_compressed(mergev.at[pl.ds(p, 16)], v, mask=mk)
                    plsc.store_compressed(mergei.at[pl.ds(p, 16)], i, mask=mk)
                    npos = _scal(plsc.all_reduce_population_count(mk))
                    return jnp.minimum(p + npos, CAP_M)
                ptr = lax.fori_loop(0, nvv, _ck, ptr)

            def _pm(j, _):
                m = jnp.max(mergev[pl.ds(j * 16, 16)])
                plsc.store_scatter(pm, [_splat_i(j)], _splat_f(m), mask=L == 0)
                return 0
            lax.fori_loop(0, 64, _pm, 0)

            def _peel(r, _):
                def _m1(p, mx):
                    return jnp.maximum(mx, jnp.max(pm[pl.ds(p * 16, 16)]))
                mx = lax.fori_loop(0, 4, _m1, jnp.float32(NEG))

                def _m2(p, carry):
                    found, jst = carry
                    pv = pm[pl.ds(p * 16, 16)]
                    mk = pv == mx
                    npos = _scal(plsc.all_reduce_population_count(mk))
                    f = _scal(plsc.all_reduce_ffs(mk))
                    jst = jnp.where((found == 0) & (npos > 0), p * 16 + f, jst)
                    return found | jnp.where(npos > 0, 1, 0), jst
                _, jst = lax.fori_loop(0, 4, _m2, (0, 0))

                cv = mergev[pl.ds(jst * 16, 16)]
                ci = mergei[pl.ds(jst * 16, 16)]
                lane = _scal(plsc.all_reduce_ffs(cv == mx))
                widx = jnp.max(jnp.where(L == lane, ci, 0))
                plsc.store_scatter(wv, [_splat_i(r)], _splat_f(mx), mask=L == 0)
                plsc.store_scatter(wi, [_splat_i(r)], _splat_i(widx), mask=L == 0)
                cv2 = jnp.where(L == lane, jnp.float32(NEG), cv)
                mergev[pl.ds(jst * 16, 16)] = cv2
                plsc.store_scatter(pm, [_splat_i(jst)], _splat_f(jnp.max(cv2)),
                                   mask=L == 0)
                return 0
            lax.fori_loop(0, K, _peel, 0)

            def _fin(i, _):
                x = wv[pl.ds(i * 16, 16)]
                outs_row[pl.ds(i * 16, 16)] = 1.0 / (1.0 + jnp.exp(-x))
                ix = wi[pl.ds(i * 16, 16)]
                q = ix // C
                outl_row[pl.ds(i * 16, 16)] = ix - q * C
                bidx[pl.ds(i * 16, 16)] = lax.shift_right_logical(q + img * N, 5)
                return 0
            lax.fori_loop(0, 8, _fin, 0)

            pltpu.sync_copy(sc.at[img], scalev)
            pltpu.async_copy(bx.at[bidx], gbox, semg).wait()
            scv = scalev[...]

            def _box(j, _):
                f = j * 16 + L
                row = lax.shift_right_logical(f, 2)
                col = f & 3
                wl = plsc.load_gather(wi, [row])
                q = wl // C
                cb = (q & 31) * 4
                c0 = cb + col - jnp.where(col >= 2, 2, 0)
                g1 = plsc.load_gather(gbox, [row, c0])
                g2 = plsc.load_gather(gbox, [row, c0 + 2])
                sgn = jnp.where(col >= 2, 0.5, -0.5).astype(jnp.float32)
                outb_row[pl.ds(j * 16, 16)] = (g1 + sgn * g2) * scv
                return 0
            lax.fori_loop(0, 32, _box, 0)

            pltpu.sync_copy(outs_row, out_s.at[img])
            pltpu.sync_copy(outl_row, out_l.at[img])
            pltpu.sync_copy(outb_row, out_b.at[img])
        return 0

    lax.fori_loop(0, 4, _image, 0)


def kernel(pred_logits, pred_boxes, target_sizes):
    flat = pred_logits.reshape(B * N * C)
    boxes2 = pred_boxes.reshape(B * N * 4 // 128, 128)
    img_h = target_sizes[:, 0].astype(jnp.float32)
    img_w = target_sizes[:, 1].astype(jnp.float32)
    scale = jnp.tile(jnp.stack([img_w, img_h, img_w, img_h], axis=1), (1, 4))

    mesh = plsc.VectorSubcoreMesh(core_axis_name="c", subcore_axis_name="s")
    out_s, out_l, out_b = pl.kernel(
        _body,
        out_type=(
            jax.ShapeDtypeStruct((B, 128), jnp.float32),
            jax.ShapeDtypeStruct((B, 128), jnp.int32),
            jax.ShapeDtypeStruct((B, 512), jnp.float32),
        ),
        mesh=mesh,
        compiler_params=pltpu.CompilerParams(needs_layout_passes=False),
        scratch_types=[
            pltpu.VMEM((WBUF,), jnp.float32),
            pltpu.VMEM((WBUF,), jnp.float32),
            pltpu.VMEM((32784,), jnp.int32),
            pltpu.VMEM((NBINS,), jnp.int32),
            pltpu.VMEM((NBINS,), jnp.int32),
            pltpu.VMEM((NBINS,), jnp.int32),
            pltpu.VMEM((CAP_T + 32,), jnp.float32),
            pltpu.VMEM((CAP_T + 32,), jnp.int32),
            pltpu.VMEM((16,), jnp.int32),
            pltpu.VMEM((16 * CAP_T,), jnp.float32),
            pltpu.VMEM((16 * CAP_T,), jnp.int32),
            pltpu.VMEM((256,), jnp.int32),
            pltpu.VMEM((CAP_M + 32,), jnp.float32),
            pltpu.VMEM((CAP_M + 32,), jnp.int32),
            pltpu.VMEM((64,), jnp.float32),
            pltpu.VMEM((128,), jnp.float32),
            pltpu.VMEM((128,), jnp.int32),
            pltpu.VMEM((128,), jnp.int32),
            pltpu.VMEM((128, 128), jnp.float32),
            pltpu.VMEM((16,), jnp.int32),
            pltpu.VMEM((128,), jnp.float32),
            pltpu.VMEM((128,), jnp.int32),
            pltpu.VMEM((512,), jnp.float32),
            pltpu.VMEM((16,), jnp.float32),
            pltpu.VMEM_SHARED((16 * NBINS,), jnp.int32),
            pltpu.VMEM_SHARED((2 * 16 * CAP_T,), jnp.float32),
            pltpu.VMEM_SHARED((2 * 16 * CAP_T,), jnp.int32),
            pltpu.VMEM_SHARED((512,), jnp.int32),
            pltpu.VMEM_SHARED((16,), jnp.int32),
            pltpu.SemaphoreType.DMA,
            pltpu.SemaphoreType.DMA,
            pltpu.SemaphoreType.DMA,
        ],
    )(flat, boxes2, scale)

    scores = out_s[:, :K]
    labels = out_l[:, :K]
    boxes = out_b.reshape(B, 128, 4)[:, :K, :]
    return scores, labels, boxes


# speedup vs baseline: 4.5270x; 1.0893x over previous
import jax
import jax.numpy as jnp
from jax import lax
from jax.experimental import pallas as pl
from jax.experimental.pallas import tpu as pltpu
from jax.experimental.pallas import tpu_sc as plsc

B = 8
N = 20000
C = 91
IMG = N * C
K = 100
SHIFT = 21
NBINS = 2048
UCH = 113744
WLEN = [1424, 1424, 1424, 1424, 1413]
WOFF = [0, 1424, 2848, 4272, 5696]
WBUF = 1424 * 16
REM_OFF = 16 * UCH
REM_V = 6
CAP_T = 512
CAP_M = 1024
LSTRIDE = 2049
NEG = -3.0e38


def _i16():
    return lax.iota(jnp.int32, 16)


def _sortkey(x):
    b = lax.bitcast_convert_type(x, jnp.int32)
    u = lax.shift_right_logical(lax.shift_right_arithmetic(b, 31), 1)
    return b ^ u


def _splat_f(s):
    return jnp.zeros((16,), jnp.float32) + s


def _splat_i(s):
    return jnp.zeros((16,), jnp.int32) + s


def _scal(v):
    return v[0]


def _body(lg, bx, sc, out_s, out_l, out_b,
          win0, win1, lhist, rhist, hbuf, thbuf, candv, candi, cntrow,
          stagev, stagei, stagec, mergev, mergei, pm, wv, wi, bidx, gbox,
          tmeta, outs_row, outl_row, outb_row, scalev,
          sh_hist, sh_candv, sh_candi, sh_cnt, sh_meta,
          sem0, sem1, semg):
    cid = lax.axis_index("c")
    sid = lax.axis_index("s")
    L = _i16()
    ones = jnp.ones((16,), jnp.int32)
    laneoff = L * LSTRIDE
    is15 = sid == 15
    wins = [win0, win1]
    sems = [sem0, sem1]

    def _image(it, _carry):
        img = cid * 4 + it
        img_off = img * IMG
        base = img_off + sid * UCH
        is_merge = sid == it
        pp = (it & 1) * 16 * CAP_T
        pc_off = (it & 1) * 256

        def _zl(i, _):
            for u in range(8):
                lhist[pl.ds((i * 8 + u) * 16, 16)] = jnp.zeros((16,), jnp.int32)
            return 0
        lax.fori_loop(0, 256, _zl, 0)
        lhist[pl.ds(2048 * 16, 16)] = jnp.zeros((16,), jnp.int32)

        descs = [pltpu.async_copy(lg.at[pl.ds(base, WLEN[0] * 16)],
                                  win0.at[pl.ds(0, WLEN[0] * 16)], sem0)]
        for w in range(5):
            descs[w].wait()
            if w + 1 < 5:
                wb = wins[(w + 1) % 2]
                nlen = WLEN[w + 1] * 16
                descs.append(pltpu.async_copy(
                    lg.at[pl.ds(base + WOFF[w + 1] * 16, nlen)],
                    wb.at[pl.ds(0, nlen)], sems[(w + 1) % 2]))
            wb = wins[w % 2]
            nmain = WLEN[w] // 8

            def _hist8(i, _):
                for u in range(8):
                    x = wb[pl.ds((i * 8 + u) * 16, 16)]
                    bkt = lax.shift_right_arithmetic(_sortkey(x), SHIFT) + 1024
                    plsc.addupdate_scatter(lhist, [laneoff + bkt], ones)
                return 0
            lax.fori_loop(0, nmain, _hist8, 0)

            def _hist1(v, _):
                x = wb[pl.ds(v * 16, 16)]
                bkt = lax.shift_right_arithmetic(_sortkey(x), SHIFT) + 1024
                plsc.addupdate_scatter(lhist, [laneoff + bkt], ones)
                return 0
            lax.fori_loop(nmain * 8, WLEN[w], _hist1, 0)

        pltpu.sync_copy(lg.at[pl.ds(img_off + REM_OFF, REM_V * 16)],
                        win0.at[pl.ds(0, REM_V * 16)])
        m15 = jnp.where(is15, ones > 0, ones < 0)
        for v in range(REM_V):
            x = win0[pl.ds(v * 16, 16)]
            bkt = lax.shift_right_arithmetic(_sortkey(x), SHIFT) + 1024
            plsc.addupdate_scatter(lhist, [laneoff + bkt], ones, mask=m15)

        def _red(r, _):
            acc = lhist[pl.ds(r * 16, 16)]
            for l in range(1, 16):
                acc = acc + lhist[pl.ds(l * LSTRIDE + r * 16, 16)]
            rhist[pl.ds(r * 16, 16)] = acc
            return 0
        lax.fori_loop(0, 128, _red, 0)
        pltpu.sync_copy(rhist, sh_hist.at[pl.ds(sid * NBINS, NBINS)])
        plsc.subcore_barrier()

        @pl.when(is_merge)
        def _():
            pltpu.sync_copy(sh_hist.at[pl.ds(0, NBINS)], thbuf)
            for t in range(1, 16):
                pltpu.sync_copy(sh_hist.at[pl.ds(t * NBINS, NBINS)], hbuf)

                def _acc(r, _):
                    for u in range(8):
                        o = (r * 8 + u) * 16
                        thbuf[pl.ds(o, 16)] = (
                            thbuf[pl.ds(o, 16)] + hbuf[pl.ds(o, 16)])
                    return 0
                lax.fori_loop(0, 16, _acc, 0)

            def _scan(rp, carry):
                total, bstar, done = carry
                r = 127 - rp
                v = thbuf[pl.ds(r * 16, 16)]
                sfx = lax.rev(jnp.cumsum(lax.rev(v, (0,))), (0,))
                mk = (sfx + total) >= K
                npos = _scal(plsc.all_reduce_population_count(mk))
                bstar = jnp.where((done == 0) & (npos > 0),
                                  r * 16 + npos - 1, bstar)
                done = done | jnp.where(npos > 0, 1, 0)
                total = total + jnp.sum(v)
                return total, bstar, done
            _, bstar, _ = lax.fori_loop(0, 128, _scan, (0, 0, 0))
            t_s = lax.shift_left(bstar - 1024, SHIFT)
            tmeta[...] = _splat_i(t_s)
            pltpu.sync_copy(tmeta, sh_meta)
        plsc.subcore_barrier()

        pltpu.sync_copy(sh_meta, tmeta)
        thr = _scal(tmeta[...])

        descs2 = [pltpu.async_copy(lg.at[pl.ds(base, WLEN[0] * 16)],
                                   win0.at[pl.ds(0, WLEN[0] * 16)], sem0)]
        cnt = 0
        for w in range(5):
            descs2[w].wait()
            if w + 1 < 5:
                wb = wins[(w + 1) % 2]
                nlen = WLEN[w + 1] * 16
                descs2.append(pltpu.async_copy(
                    lg.at[pl.ds(base + WOFF[w + 1] * 16, nlen)],
                    wb.at[pl.ds(0, nlen)], sems[(w + 1) % 2]))
            wb = wins[w % 2]
            gbase = sid * UCH + WOFF[w] * 16
            nmain = WLEN[w] // 8

            def _coll8(i, c):
                for u in range(8):
                    v = i * 8 + u
                    x = wb[pl.ds(v * 16, 16)]
                    mk = _sortkey(x) >= thr
                    gi = gbase + v * 16 + L
                    plsc.store_compressed(candv.at[pl.ds(c, 16)], x, mask=mk)
                    plsc.store_compressed(candi.at[pl.ds(c, 16)], gi, mask=mk)
                    npos = _scal(plsc.all_reduce_population_count(mk))
                    c = jnp.minimum(c + npos, CAP_T)
                return c
            cnt = lax.fori_loop(0, nmain, _coll8, cnt)

            def _coll1(v, c):
                x = wb[pl.ds(v * 16, 16)]
                mk = _sortkey(x) >= thr
                gi = gbase + v * 16 + L
                plsc.store_compressed(candv.at[pl.ds(c, 16)], x, mask=mk)
                plsc.store_compressed(candi.at[pl.ds(c, 16)], gi, mask=mk)
                npos = _scal(plsc.all_reduce_population_count(mk))
                return jnp.minimum(c + npos, CAP_T)
            cnt = lax.fori_loop(nmain * 8, WLEN[w], _coll1, cnt)

        pltpu.sync_copy(lg.at[pl.ds(img_off + REM_OFF, REM_V * 16)],
                        win0.at[pl.ds(0, REM_V * 16)])
        for v in range(REM_V):
            x = win0[pl.ds(v * 16, 16)]
            mk = (_sortkey(x) >= thr) & m15
            gi = REM_OFF + v * 16 + L
            plsc.store_compressed(candv.at[pl.ds(cnt, 16)], x, mask=mk)
            plsc.store_compressed(candi.at[pl.ds(cnt, 16)], gi, mask=mk)
            npos = _scal(plsc.all_reduce_population_count(mk))
            cnt = jnp.minimum(cnt + npos, CAP_T)

        cntrow[...] = _splat_i(cnt)
        pltpu.sync_copy(candv.at[pl.ds(0, CAP_T)],
                        sh_candv.at[pl.ds(pp + sid * CAP_T, CAP_T)])
        pltpu.sync_copy(candi.at[pl.ds(0, CAP_T)],
                        sh_candi.at[pl.ds(pp + sid * CAP_T, CAP_T)])
        pltpu.sync_copy(cntrow, sh_cnt.at[pl.ds(pc_off + sid * 16, 16)])
        plsc.subcore_barrier()

        @pl.when(is_merge)
        def _():
            pltpu.sync_copy(sh_candv.at[pl.ds(pp, 16 * CAP_T)], stagev)
            pltpu.sync_copy(sh_candi.at[pl.ds(pp, 16 * CAP_T)], stagei)
            pltpu.sync_copy(sh_cnt.at[pl.ds(pc_off, 256)], stagec)

            def _pre(i, _):
                mergev[pl.ds(i * 16, 16)] = _splat_f(NEG)
                return 0
            lax.fori_loop(0, 66, _pre, 0)

            def _prew(i, _):
                wv[pl.ds(i * 16, 16)] = _splat_f(-20.0)
                wi[pl.ds(i * 16, 16)] = jnp.zeros((16,), jnp.int32)
                bidx[pl.ds(i * 16, 16)] = jnp.zeros((16,), jnp.int32)
                return 0
            lax.fori_loop(0, 8, _prew, 0)

            ptr = 0
            for t in range(16):
                nt = _scal(stagec[pl.ds(t * 16, 16)])
                nvv = lax.shift_right_logical(nt + 15, 4)

                def _ck(k, p):
                    v = stagev[pl.ds(t * CAP_T + k * 16, 16)]
                    i = stagei[pl.ds(t * CAP_T + k * 16, 16)]
                    mk = L < (nt - k * 16)
                    plsc.store_compressed(mergev.at[pl.ds(p, 16)], v, mask=mk)
                    plsc.store_compressed(mergei.at[pl.ds(p, 16)], i, mask=mk)
                    npos = _scal(plsc.all_reduce_population_count(mk))
                    return jnp.minimum(p + npos, CAP_M)
                ptr = lax.fori_loop(0, nvv, _ck, ptr)

            def _pm(j, _):
                m = jnp.max(mergev[pl.ds(j * 16, 16)])
                plsc.store_scatter(pm, [_splat_i(j)], _splat_f(m), mask=L == 0)
                return 0
            lax.fori_loop(0, 64, _pm, 0)

            def _peel(r, _):
                def _m1(p, mx):
                    return jnp.maximum(mx, jnp.max(pm[pl.ds(p * 16, 16)]))
                mx = lax.fori_loop(0, 4, _m1, jnp.float32(NEG))

                def _m2(p, carry):
                    found, jst = carry
                    pv = pm[pl.ds(p * 16, 16)]
                    mk = pv == mx
                    npos = _scal(plsc.all_reduce_population_count(mk))
                    f = _scal(plsc.all_reduce_ffs(mk))
                    jst = jnp.where((found == 0) & (npos > 0), p * 16 + f, jst)
                    return found | jnp.where(npos > 0, 1, 0), jst
                _, jst = lax.fori_loop(0, 4, _m2, (0, 0))

                cv = mergev[pl.ds(jst * 16, 16)]
                ci = mergei[pl.ds(jst * 16, 16)]
                lane = _scal(plsc.all_reduce_ffs(cv == mx))
                widx = jnp.max(jnp.where(L == lane, ci, 0))
                plsc.store_scatter(wv, [_splat_i(r)], _splat_f(mx), mask=L == 0)
                plsc.store_scatter(wi, [_splat_i(r)], _splat_i(widx), mask=L == 0)
                cv2 = jnp.where(L == lane, jnp.float32(NEG), cv)
                mergev[pl.ds(jst * 16, 16)] = cv2
                plsc.store_scatter(pm, [_splat_i(jst)], _splat_f(jnp.max(cv2)),
                                   mask=L == 0)
                return 0
            lax.fori_loop(0, K, _peel, 0)

            def _fin(i, _):
                x = wv[pl.ds(i * 16, 16)]
                outs_row[pl.ds(i * 16, 16)] = 1.0 / (1.0 + jnp.exp(-x))
                ix = wi[pl.ds(i * 16, 16)]
                q = ix // C
                outl_row[pl.ds(i * 16, 16)] = ix - q * C
                bidx[pl.ds(i * 16, 16)] = lax.shift_right_logical(q + img * N, 5)
                return 0
            lax.fori_loop(0, 8, _fin, 0)

            pltpu.sync_copy(sc.at[img], scalev)
            pltpu.async_copy(bx.at[bidx], gbox, semg).wait()
            scv = scalev[...]

            def _box(j, _):
                f = j * 16 + L
                row = lax.shift_right_logical(f, 2)
                col = f & 3
                wl = plsc.load_gather(wi, [row])
                q = wl // C
                cb = (q & 31) * 4
                c0 = cb + col - jnp.where(col >= 2, 2, 0)
                g1 = plsc.load_gather(gbox, [row, c0])
                g2 = plsc.load_gather(gbox, [row, c0 + 2])
                sgn = jnp.where(col >= 2, 0.5, -0.5).astype(jnp.float32)
                outb_row[pl.ds(j * 16, 16)] = (g1 + sgn * g2) * scv
                return 0
            lax.fori_loop(0, 32, _box, 0)

            pltpu.sync_copy(outs_row, out_s.at[img])
            pltpu.sync_copy(outl_row, out_l.at[img])
            pltpu.sync_copy(outb_row, out_b.at[img])
        return 0

    lax.fori_loop(0, 4, _image, 0)


def kernel(pred_logits, pred_boxes, target_sizes):
    flat = pred_logits.reshape(B * N * C)
    boxes2 = pred_boxes.reshape(B * N * 4 // 128, 128)
    img_h = target_sizes[:, 0].astype(jnp.float32)
    img_w = target_sizes[:, 1].astype(jnp.float32)
    scale = jnp.tile(jnp.stack([img_w, img_h, img_w, img_h], axis=1), (1, 4))

    mesh = plsc.VectorSubcoreMesh(core_axis_name="c", subcore_axis_name="s")
    out_s, out_l, out_b = pl.kernel(
        _body,
        out_type=(
            jax.ShapeDtypeStruct((B, 128), jnp.float32),
            jax.ShapeDtypeStruct((B, 128), jnp.int32),
            jax.ShapeDtypeStruct((B, 512), jnp.float32),
        ),
        mesh=mesh,
        compiler_params=pltpu.CompilerParams(needs_layout_passes=False),
        scratch_types=[
            pltpu.VMEM((WBUF,), jnp.float32),
            pltpu.VMEM((WBUF,), jnp.float32),
            pltpu.VMEM((32784,), jnp.int32),
            pltpu.VMEM((NBINS,), jnp.int32),
            pltpu.VMEM((NBINS,), jnp.int32),
            pltpu.VMEM((NBINS,), jnp.int32),
            pltpu.VMEM((CAP_T + 32,), jnp.float32),
            pltpu.VMEM((CAP_T + 32,), jnp.int32),
            pltpu.VMEM((16,), jnp.int32),
            pltpu.VMEM((16 * CAP_T,), jnp.float32),
            pltpu.VMEM((16 * CAP_T,), jnp.int32),
            pltpu.VMEM((256,), jnp.int32),
            pltpu.VMEM((CAP_M + 32,), jnp.float32),
            pltpu.VMEM((CAP_M + 32,), jnp.int32),
            pltpu.VMEM((64,), jnp.float32),
            pltpu.VMEM((128,), jnp.float32),
            pltpu.VMEM((128,), jnp.int32),
            pltpu.VMEM((128,), jnp.int32),
            pltpu.VMEM((128, 128), jnp.float32),
            pltpu.VMEM((16,), jnp.int32),
            pltpu.VMEM((128,), jnp.float32),
            pltpu.VMEM((128,), jnp.int32),
            pltpu.VMEM((512,), jnp.float32),
            pltpu.VMEM((16,), jnp.float32),
            pltpu.VMEM_SHARED((16 * NBINS,), jnp.int32),
            pltpu.VMEM_SHARED((2 * 16 * CAP_T,), jnp.float32),
            pltpu.VMEM_SHARED((2 * 16 * CAP_T,), jnp.int32),
            pltpu.VMEM_SHARED((512,), jnp.int32),
            pltpu.VMEM_SHARED((16,), jnp.int32),
            pltpu.SemaphoreType.DMA,
            pltpu.SemaphoreType.DMA,
            pltpu.SemaphoreType.DMA,
        ],
    )(flat, boxes2, scale)

    scores = out_s[:, :K]
    labels = out_l[:, :K]
    boxes = out_b.reshape(B, 128, 4)[:, :K, :]
    return scores, labels, boxes


# speedup vs baseline: 8.3592x; 1.8465x over previous
import jax
import jax.numpy as jnp
from jax import lax
from jax.experimental import pallas as pl
from jax.experimental.pallas import tpu as pltpu
from jax.experimental.pallas import tpu_sc as plsc

B = 8
N = 20000
C = 91
IMG = N * C
K = 100
SHIFT = 21
NBINS = 2048
UCH = 113744
WLEN = [1424, 1424, 1424, 1424, 1413]
WOFF = [0, 1424, 2848, 4272, 5696]
WBUF = 1424 * 16
REM_OFF = 16 * UCH
REM_V = 6
CAP_T = 512
CAP_M = 1024
LSTRIDE = 2049
NEG = -3.0e38


def _i16():
    return lax.iota(jnp.int32, 16)


def _sortkey(x):
    b = lax.bitcast_convert_type(x, jnp.int32)
    u = lax.shift_right_logical(lax.shift_right_arithmetic(b, 31), 1)
    return b ^ u


def _splat_f(s):
    return jnp.zeros((16,), jnp.float32) + s


def _splat_i(s):
    return jnp.zeros((16,), jnp.int32) + s


def _scal(v):
    return v[0]


def _body(lg, bx, sc, out_s, out_l, out_b,
          win0, win1, lhist, rhist, hbuf, thbuf, candv, candi, cntrow,
          stagev, stagei, stagec, mergev, mergei, pm, wv, wi, bidx, gbox,
          tmeta, outs_row, outl_row, outb_row, scalev,
          sh_hist, sh_candv, sh_candi, sh_cnt, sh_meta,
          sem0, sem1, semg):
    cid = lax.axis_index("c")
    sid = lax.axis_index("s")
    L = _i16()
    ones = jnp.ones((16,), jnp.int32)
    laneoff = L * LSTRIDE
    is15 = sid == 15
    wins = [win0, win1]
    sems = [sem0, sem1]

    def _image(it, _carry):
        img = cid * 4 + it
        img_off = img * IMG
        base = img_off + sid * UCH
        is_merge = sid == it
        pp = (it & 1) * 16 * CAP_T
        pc_off = (it & 1) * 256

        def _zl(i, _):
            for u in range(8):
                lhist[pl.ds((i * 8 + u) * 16, 16)] = jnp.zeros((16,), jnp.int32)
            return 0
        lax.fori_loop(0, 256, _zl, 0)
        lhist[pl.ds(2048 * 16, 16)] = jnp.zeros((16,), jnp.int32)

        descs = [pltpu.async_copy(lg.at[pl.ds(base, WLEN[0] * 16)],
                                  win0.at[pl.ds(0, WLEN[0] * 16)], sem0)]
        for w in range(5):
            descs[w].wait()
            if w + 1 < 5:
                wb = wins[(w + 1) % 2]
                nlen = WLEN[w + 1] * 16
                descs.append(pltpu.async_copy(
                    lg.at[pl.ds(base + WOFF[w + 1] * 16, nlen)],
                    wb.at[pl.ds(0, nlen)], sems[(w + 1) % 2]))
            wb = wins[w % 2]

            @plsc.parallel_loop(0, WLEN[w], unroll=8)
            def _hist(v):
                x = wb[pl.ds(v * 16, 16)]
                bkt = lax.shift_right_arithmetic(_sortkey(x), SHIFT) + 1024
                plsc.addupdate_scatter(lhist, [laneoff + bkt], ones)

        pltpu.sync_copy(lg.at[pl.ds(img_off + REM_OFF, REM_V * 16)],
                        win0.at[pl.ds(0, REM_V * 16)])
        m15 = jnp.where(is15, ones > 0, ones < 0)
        for v in range(REM_V):
            x = win0[pl.ds(v * 16, 16)]
            bkt = lax.shift_right_arithmetic(_sortkey(x), SHIFT) + 1024
            plsc.addupdate_scatter(lhist, [laneoff + bkt], ones, mask=m15)

        def _red(r, _):
            acc = lhist[pl.ds(r * 16, 16)]
            for l in range(1, 16):
                acc = acc + lhist[pl.ds(l * LSTRIDE + r * 16, 16)]
            rhist[pl.ds(r * 16, 16)] = acc
            return 0
        lax.fori_loop(0, 128, _red, 0)
        pltpu.sync_copy(rhist, sh_hist.at[pl.ds(sid * NBINS, NBINS)])
        plsc.subcore_barrier()

        @pl.when(is_merge)
        def _():
            pltpu.sync_copy(sh_hist.at[pl.ds(0, NBINS)], thbuf)
            for t in range(1, 16):
                pltpu.sync_copy(sh_hist.at[pl.ds(t * NBINS, NBINS)], hbuf)

                def _acc(r, _):
                    for u in range(8):
                        o = (r * 8 + u) * 16
                        thbuf[pl.ds(o, 16)] = (
                            thbuf[pl.ds(o, 16)] + hbuf[pl.ds(o, 16)])
                    return 0
                lax.fori_loop(0, 16, _acc, 0)

            def _scan(rp, carry):
                total, bstar, done = carry
                r = 127 - rp
                v = thbuf[pl.ds(r * 16, 16)]
                sfx = lax.rev(jnp.cumsum(lax.rev(v, (0,))), (0,))
                mk = (sfx + total) >= K
                npos = _scal(plsc.all_reduce_population_count(mk))
                bstar = jnp.where((done == 0) & (npos > 0),
                                  r * 16 + npos - 1, bstar)
                done = done | jnp.where(npos > 0, 1, 0)
                total = total + jnp.sum(v)
                return total, bstar, done
            _, bstar, _ = lax.fori_loop(0, 128, _scan, (0, 0, 0))
            t_s = lax.shift_left(bstar - 1024, SHIFT)
            tmeta[...] = _splat_i(t_s)
            pltpu.sync_copy(tmeta, sh_meta)
        plsc.subcore_barrier()

        pltpu.sync_copy(sh_meta, tmeta)
        thr = _scal(tmeta[...])

        descs2 = [pltpu.async_copy(lg.at[pl.ds(base, WLEN[0] * 16)],
                                   win0.at[pl.ds(0, WLEN[0] * 16)], sem0)]
        cnt = 0
        for w in range(5):
            descs2[w].wait()
            if w + 1 < 5:
                wb = wins[(w + 1) % 2]
                nlen = WLEN[w + 1] * 16
                descs2.append(pltpu.async_copy(
                    lg.at[pl.ds(base + WOFF[w + 1] * 16, nlen)],
                    wb.at[pl.ds(0, nlen)], sems[(w + 1) % 2]))
            wb = wins[w % 2]
            gbase = sid * UCH + WOFF[w] * 16

            @plsc.parallel_loop(0, WLEN[w], unroll=8, carry=jnp.int32(0) + cnt)
            def _coll(v, c):
                x = wb[pl.ds(v * 16, 16)]
                mk = _sortkey(x) >= thr
                gi = gbase + v * 16 + L
                plsc.store_compressed(candv.at[pl.ds(c, 16)], x, mask=mk)
                plsc.store_compressed(candi.at[pl.ds(c, 16)], gi, mask=mk)
                npos = _scal(plsc.all_reduce_population_count(mk))
                return jnp.minimum(c + npos, CAP_T)
            cnt = _coll

        pltpu.sync_copy(lg.at[pl.ds(img_off + REM_OFF, REM_V * 16)],
                        win0.at[pl.ds(0, REM_V * 16)])
        for v in range(REM_V):
            x = win0[pl.ds(v * 16, 16)]
            mk = (_sortkey(x) >= thr) & m15
            gi = REM_OFF + v * 16 + L
            plsc.store_compressed(candv.at[pl.ds(cnt, 16)], x, mask=mk)
            plsc.store_compressed(candi.at[pl.ds(cnt, 16)], gi, mask=mk)
            npos = _scal(plsc.all_reduce_population_count(mk))
            cnt = jnp.minimum(cnt + npos, CAP_T)

        cntrow[...] = _splat_i(cnt)
        pltpu.sync_copy(candv.at[pl.ds(0, CAP_T)],
                        sh_candv.at[pl.ds(pp + sid * CAP_T, CAP_T)])
        pltpu.sync_copy(candi.at[pl.ds(0, CAP_T)],
                        sh_candi.at[pl.ds(pp + sid * CAP_T, CAP_T)])
        pltpu.sync_copy(cntrow, sh_cnt.at[pl.ds(pc_off + sid * 16, 16)])
        plsc.subcore_barrier()

        @pl.when(is_merge)
        def _():
            pltpu.sync_copy(sh_candv.at[pl.ds(pp, 16 * CAP_T)], stagev)
            pltpu.sync_copy(sh_candi.at[pl.ds(pp, 16 * CAP_T)], stagei)
            pltpu.sync_copy(sh_cnt.at[pl.ds(pc_off, 256)], stagec)

            def _pre(i, _):
                mergev[pl.ds(i * 16, 16)] = _splat_f(NEG)
                return 0
            lax.fori_loop(0, 66, _pre, 0)

            def _prew(i, _):
                wv[pl.ds(i * 16, 16)] = _splat_f(-20.0)
                wi[pl.ds(i * 16, 16)] = jnp.zeros((16,), jnp.int32)
                bidx[pl.ds(i * 16, 16)] = jnp.zeros((16,), jnp.int32)
                return 0
            lax.fori_loop(0, 8, _prew, 0)

            ptr = 0
            for t in range(16):
                nt = _scal(stagec[pl.ds(t * 16, 16)])
                nvv = lax.shift_right_logical(nt + 15, 4)

                def _ck(k, p):
                    v = stagev[pl.ds(t * CAP_T + k * 16, 16)]
                    i = stagei[pl.ds(t * CAP_T + k * 16, 16)]
                    mk = L < (nt - k * 16)
                    plsc.store_compressed(mergev.at[pl.ds(p, 16)], v, mask=mk)
                    plsc.store_compressed(mergei.at[pl.ds(p, 16)], i, mask=mk)
                    npos = _scal(plsc.all_reduce_population_count(mk))
                    return jnp.minimum(p + npos, CAP_M)
                ptr = lax.fori_loop(0, nvv, _ck, ptr)

            def _pm(j, _):
                m = jnp.max(mergev[pl.ds(j * 16, 16)])
                plsc.store_scatter(pm, [_splat_i(j)], _splat_f(m), mask=L == 0)
                return 0
            lax.fori_loop(0, 64, _pm, 0)

            def _peel(r, _):
                def _m1(p, mx):
                    return jnp.maximum(mx, jnp.max(pm[pl.ds(p * 16, 16)]))
                mx = lax.fori_loop(0, 4, _m1, jnp.float32(NEG))

                def _m2(p, carry):
                    found, jst = carry
                    pv = pm[pl.ds(p * 16, 16)]
                    mk = pv == mx
                    npos = _scal(plsc.all_reduce_population_count(mk))
                    f = _scal(plsc.all_reduce_ffs(mk))
                    jst = jnp.where((found == 0) & (npos > 0), p * 16 + f, jst)
                    return found | jnp.where(npos > 0, 1, 0), jst
                _, jst = lax.fori_loop(0, 4, _m2, (0, 0))

                cv = mergev[pl.ds(jst * 16, 16)]
                ci = mergei[pl.ds(jst * 16, 16)]
                lane = _scal(plsc.all_reduce_ffs(cv == mx))
                widx = jnp.max(jnp.where(L == lane, ci, 0))
                plsc.store_scatter(wv, [_splat_i(r)], _splat_f(mx), mask=L == 0)
                plsc.store_scatter(wi, [_splat_i(r)], _splat_i(widx), mask=L == 0)
                cv2 = jnp.where(L == lane, jnp.float32(NEG), cv)
                mergev[pl.ds(jst * 16, 16)] = cv2
                plsc.store_scatter(pm, [_splat_i(jst)], _splat_f(jnp.max(cv2)),
                                   mask=L == 0)
                return 0
            lax.fori_loop(0, K, _peel, 0)

            def _fin(i, _):
                x = wv[pl.ds(i * 16, 16)]
                outs_row[pl.ds(i * 16, 16)] = 1.0 / (1.0 + jnp.exp(-x))
                ix = wi[pl.ds(i * 16, 16)]
                q = ix // C
                outl_row[pl.ds(i * 16, 16)] = ix - q * C
                bidx[pl.ds(i * 16, 16)] = lax.shift_right_logical(q + img * N, 5)
                return 0
            lax.fori_loop(0, 8, _fin, 0)

            pltpu.sync_copy(sc.at[img], scalev)
            pltpu.async_copy(bx.at[bidx], gbox, semg).wait()
            scv = scalev[...]

            def _box(j, _):
                f = j * 16 + L
                row = lax.shift_right_logical(f, 2)
                col = f & 3
                wl = plsc.load_gather(wi, [row])
                q = wl // C
                cb = (q & 31) * 4
                c0 = cb + col - jnp.where(col >= 2, 2, 0)
                g1 = plsc.load_gather(gbox, [row, c0])
                g2 = plsc.load_gather(gbox, [row, c0 + 2])
                sgn = jnp.where(col >= 2, 0.5, -0.5).astype(jnp.float32)
                outb_row[pl.ds(j * 16, 16)] = (g1 + sgn * g2) * scv
                return 0
            lax.fori_loop(0, 32, _box, 0)

            pltpu.sync_copy(outs_row, out_s.at[img])
            pltpu.sync_copy(outl_row, out_l.at[img])
            pltpu.sync_copy(outb_row, out_b.at[img])
        return 0

    lax.fori_loop(0, 4, _image, 0)


def kernel(pred_logits, pred_boxes, target_sizes):
    flat = pred_logits.reshape(B * N * C)
    boxes2 = pred_boxes.reshape(B * N * 4 // 128, 128)
    img_h = target_sizes[:, 0].astype(jnp.float32)
    img_w = target_sizes[:, 1].astype(jnp.float32)
    scale = jnp.tile(jnp.stack([img_w, img_h, img_w, img_h], axis=1), (1, 4))

    mesh = plsc.VectorSubcoreMesh(core_axis_name="c", subcore_axis_name="s")
    out_s, out_l, out_b = pl.kernel(
        _body,
        out_type=(
            jax.ShapeDtypeStruct((B, 128), jnp.float32),
            jax.ShapeDtypeStruct((B, 128), jnp.int32),
            jax.ShapeDtypeStruct((B, 512), jnp.float32),
        ),
        mesh=mesh,
        compiler_params=pltpu.CompilerParams(needs_layout_passes=False),
        scratch_types=[
            pltpu.VMEM((WBUF,), jnp.float32),
            pltpu.VMEM((WBUF,), jnp.float32),
            pltpu.VMEM((32784,), jnp.int32),
            pltpu.VMEM((NBINS,), jnp.int32),
            pltpu.VMEM((NBINS,), jnp.int32),
            pltpu.VMEM((NBINS,), jnp.int32),
            pltpu.VMEM((CAP_T + 32,), jnp.float32),
            pltpu.VMEM((CAP_T + 32,), jnp.int32),
            pltpu.VMEM((16,), jnp.int32),
            pltpu.VMEM((16 * CAP_T,), jnp.float32),
            pltpu.VMEM((16 * CAP_T,), jnp.int32),
            pltpu.VMEM((256,), jnp.int32),
            pltpu.VMEM((CAP_M + 32,), jnp.float32),
            pltpu.VMEM((CAP_M + 32,), jnp.int32),
            pltpu.VMEM((64,), jnp.float32),
            pltpu.VMEM((128,), jnp.float32),
            pltpu.VMEM((128,), jnp.int32),
            pltpu.VMEM((128,), jnp.int32),
            pltpu.VMEM((128, 128), jnp.float32),
            pltpu.VMEM((16,), jnp.int32),
            pltpu.VMEM((128,), jnp.float32),
            pltpu.VMEM((128,), jnp.int32),
            pltpu.VMEM((512,), jnp.float32),
            pltpu.VMEM((16,), jnp.float32),
            pltpu.VMEM_SHARED((16 * NBINS,), jnp.int32),
            pltpu.VMEM_SHARED((2 * 16 * CAP_T,), jnp.float32),
            pltpu.VMEM_SHARED((2 * 16 * CAP_T,), jnp.int32),
            pltpu.VMEM_SHARED((512,), jnp.int32),
            pltpu.VMEM_SHARED((16,), jnp.int32),
            pltpu.SemaphoreType.DMA,
            pltpu.SemaphoreType.DMA,
            pltpu.SemaphoreType.DMA,
        ],
    )(flat, boxes2, scale)

    scores = out_s[:, :K]
    labels = out_l[:, :K]
    boxes = out_b.reshape(B, 128, 4)[:, :K, :]
    return scores, labels, boxes


# speedup vs baseline: 8.4200x; 1.0073x over previous
import jax
import jax.numpy as jnp
from jax import lax
from jax.experimental import pallas as pl
from jax.experimental.pallas import tpu as pltpu
from jax.experimental.pallas import tpu_sc as plsc

B = 8
N = 20000
C = 91
IMG = N * C
K = 100
SHIFT = 21
NBINS = 2048
UCH = 113744
WLEN = [1424, 1424, 1424, 1424, 1413]
WOFF = [0, 1424, 2848, 4272, 5696]
WBUF = 1424 * 16
REM_OFF = 16 * UCH
REM_V = 6
CAP_T = 512
CAP_M = 1024
LSTRIDE = 2049
NEG = -3.0e38


def _i16():
    return lax.iota(jnp.int32, 16)


def _sortkey(x):
    b = lax.bitcast_convert_type(x, jnp.int32)
    u = lax.shift_right_logical(lax.shift_right_arithmetic(b, 31), 1)
    return b ^ u


def _splat_f(s):
    return jnp.zeros((16,), jnp.float32) + s


def _splat_i(s):
    return jnp.zeros((16,), jnp.int32) + s


def _scal(v):
    return v[0]


def _body(lg, bx, sc, out_s, out_l, out_b,
          win0, win1, lhist, rhist, hbuf, thbuf, candv, candi, cntrow,
          stagev, stagei, stagec, mergev, mergei, pm, wv, wi, bidx, gbox,
          tmeta, outs_row, outl_row, outb_row, scalev,
          sh_hist, sh_candv, sh_candi, sh_cnt, sh_meta,
          sem0, sem1, semg):
    cid = lax.axis_index("c")
    sid = lax.axis_index("s")
    L = _i16()
    ones = jnp.ones((16,), jnp.int32)
    laneoff = L * LSTRIDE
    is15 = sid == 15
    wins = [win0, win1]
    sems = [sem0, sem1]

    def _image(it, _carry):
        img = cid * 4 + it
        img_off = img * IMG
        base = img_off + sid * UCH
        is_merge = sid == it
        pp = (it & 1) * 16 * CAP_T
        pc_off = (it & 1) * 256

        @plsc.parallel_loop(0, 2049, unroll=8)
        def _zl(i):
            lhist[pl.ds(i * 16, 16)] = jnp.zeros((16,), jnp.int32)

        descs = [pltpu.async_copy(lg.at[pl.ds(base, WLEN[0] * 16)],
                                  win0.at[pl.ds(0, WLEN[0] * 16)], sem0)]
        for w in range(5):
            descs[w].wait()
            if w + 1 < 5:
                wb = wins[(w + 1) % 2]
                nlen = WLEN[w + 1] * 16
                descs.append(pltpu.async_copy(
                    lg.at[pl.ds(base + WOFF[w + 1] * 16, nlen)],
                    wb.at[pl.ds(0, nlen)], sems[(w + 1) % 2]))
            wb = wins[w % 2]

            @plsc.parallel_loop(0, WLEN[w], unroll=8)
            def _hist(v):
                x = wb[pl.ds(v * 16, 16)]
                bkt = lax.shift_right_arithmetic(_sortkey(x), SHIFT) + 1024
                plsc.addupdate_scatter(lhist, [laneoff + bkt], ones)

        pltpu.sync_copy(lg.at[pl.ds(img_off + REM_OFF, REM_V * 16)],
                        win0.at[pl.ds(0, REM_V * 16)])
        m15 = jnp.where(is15, ones > 0, ones < 0)
        for v in range(REM_V):
            x = win0[pl.ds(v * 16, 16)]
            bkt = lax.shift_right_arithmetic(_sortkey(x), SHIFT) + 1024
            plsc.addupdate_scatter(lhist, [laneoff + bkt], ones, mask=m15)

        @plsc.parallel_loop(0, 128, unroll=2)
        def _red(r):
            acc = lhist[pl.ds(r * 16, 16)]
            for l in range(1, 16):
                acc = acc + lhist[pl.ds(l * LSTRIDE + r * 16, 16)]
            rhist[pl.ds(r * 16, 16)] = acc
        pltpu.sync_copy(rhist, sh_hist.at[pl.ds(sid * NBINS, NBINS)])
        plsc.subcore_barrier()

        @pl.when(is_merge)
        def _():
            pltpu.sync_copy(sh_hist.at[pl.ds(0, NBINS)], thbuf)
            for t in range(1, 16):
                pltpu.sync_copy(sh_hist.at[pl.ds(t * NBINS, NBINS)], hbuf)

                @plsc.parallel_loop(0, 128, unroll=8)
                def _acc(r):
                    thbuf[pl.ds(r * 16, 16)] = (
                        thbuf[pl.ds(r * 16, 16)] + hbuf[pl.ds(r * 16, 16)])

            def _scan(rp, carry):
                total, bstar, done = carry
                r = 127 - rp
                v = thbuf[pl.ds(r * 16, 16)]
                sfx = lax.rev(jnp.cumsum(lax.rev(v, (0,))), (0,))
                mk = (sfx + total) >= K
                npos = _scal(plsc.all_reduce_population_count(mk))
                bstar = jnp.where((done == 0) & (npos > 0),
                                  r * 16 + npos - 1, bstar)
                done = done | jnp.where(npos > 0, 1, 0)
                total = total + jnp.sum(v)
                return total, bstar, done
            _, bstar, _ = lax.fori_loop(0, 128, _scan, (0, 0, 0))
            t_s = lax.shift_left(bstar - 1024, SHIFT)
            tmeta[...] = _splat_i(t_s)
            pltpu.sync_copy(tmeta, sh_meta)
        plsc.subcore_barrier()

        pltpu.sync_copy(sh_meta, tmeta)
        thr = _scal(tmeta[...])

        descs2 = [pltpu.async_copy(lg.at[pl.ds(base, WLEN[0] * 16)],
                                   win0.at[pl.ds(0, WLEN[0] * 16)], sem0)]
        cnt = 0
        for w in range(5):
            descs2[w].wait()
            if w + 1 < 5:
                wb = wins[(w + 1) % 2]
                nlen = WLEN[w + 1] * 16
                descs2.append(pltpu.async_copy(
                    lg.at[pl.ds(base + WOFF[w + 1] * 16, nlen)],
                    wb.at[pl.ds(0, nlen)], sems[(w + 1) % 2]))
            wb = wins[w % 2]
            gbase = sid * UCH + WOFF[w] * 16

            @plsc.parallel_loop(0, WLEN[w], unroll=8, carry=jnp.int32(0) + cnt)
            def _coll(v, c):
                x = wb[pl.ds(v * 16, 16)]
                mk = _sortkey(x) >= thr
                gi = gbase + v * 16 + L
                plsc.store_compressed(candv.at[pl.ds(c, 16)], x, mask=mk)
                plsc.store_compressed(candi.at[pl.ds(c, 16)], gi, mask=mk)
                npos = _scal(plsc.all_reduce_population_count(mk))
                return jnp.minimum(c + npos, CAP_T)
            cnt = _coll

        pltpu.sync_copy(lg.at[pl.ds(img_off + REM_OFF, REM_V * 16)],
                        win0.at[pl.ds(0, REM_V * 16)])
        for v in range(REM_V):
            x = win0[pl.ds(v * 16, 16)]
            mk = (_sortkey(x) >= thr) & m15
            gi = REM_OFF + v * 16 + L
            plsc.store_compressed(candv.at[pl.ds(cnt, 16)], x, mask=mk)
            plsc.store_compressed(candi.at[pl.ds(cnt, 16)], gi, mask=mk)
            npos = _scal(plsc.all_reduce_population_count(mk))
            cnt = jnp.minimum(cnt + npos, CAP_T)

        cntrow[...] = _splat_i(cnt)
        pltpu.sync_copy(candv.at[pl.ds(0, CAP_T)],
                        sh_candv.at[pl.ds(pp + sid * CAP_T, CAP_T)])
        pltpu.sync_copy(candi.at[pl.ds(0, CAP_T)],
                        sh_candi.at[pl.ds(pp + sid * CAP_T, CAP_T)])
        pltpu.sync_copy(cntrow, sh_cnt.at[pl.ds(pc_off + sid * 16, 16)])
        plsc.subcore_barrier()

        @pl.when(is_merge)
        def _():
            pltpu.sync_copy(sh_candv.at[pl.ds(pp, 16 * CAP_T)], stagev)
            pltpu.sync_copy(sh_candi.at[pl.ds(pp, 16 * CAP_T)], stagei)
            pltpu.sync_copy(sh_cnt.at[pl.ds(pc_off, 256)], stagec)

            @plsc.parallel_loop(0, 66, unroll=4)
            def _pre(i):
                mergev[pl.ds(i * 16, 16)] = _splat_f(NEG)

            @plsc.parallel_loop(0, 8, unroll=4)
            def _prew(i):
                wv[pl.ds(i * 16, 16)] = _splat_f(-20.0)
                wi[pl.ds(i * 16, 16)] = jnp.zeros((16,), jnp.int32)
                bidx[pl.ds(i * 16, 16)] = jnp.zeros((16,), jnp.int32)

            ptr = 0
            for t in range(16):
                nt = _scal(stagec[pl.ds(t * 16, 16)])
                nvv = lax.shift_right_logical(nt + 15, 4)

                def _ck(k, p):
                    v = stagev[pl.ds(t * CAP_T + k * 16, 16)]
                    i = stagei[pl.ds(t * CAP_T + k * 16, 16)]
                    mk = L < (nt - k * 16)
                    plsc.store_compressed(mergev.at[pl.ds(p, 16)], v, mask=mk)
                    plsc.store_compressed(mergei.at[pl.ds(p, 16)], i, mask=mk)
                    npos = _scal(plsc.all_reduce_population_count(mk))
                    return jnp.minimum(p + npos, CAP_M)
                ptr = lax.fori_loop(0, nvv, _ck, ptr)

            @plsc.parallel_loop(0, 64, unroll=4)
            def _pm(j):
                m = jnp.max(mergev[pl.ds(j * 16, 16)])
                plsc.store_scatter(pm, [_splat_i(j)], _splat_f(m), mask=L == 0)

            def _peel(r, _):
                def _m1(p, mx):
                    return jnp.maximum(mx, jnp.max(pm[pl.ds(p * 16, 16)]))
                mx = lax.fori_loop(0, 4, _m1, jnp.float32(NEG))

                def _m2(p, carry):
                    found, jst = carry
                    pv = pm[pl.ds(p * 16, 16)]
                    mk = pv == mx
                    npos = _scal(plsc.all_reduce_population_count(mk))
                    f = _scal(plsc.all_reduce_ffs(mk))
                    jst = jnp.where((found == 0) & (npos > 0), p * 16 + f, jst)
                    return found | jnp.where(npos > 0, 1, 0), jst
                _, jst = lax.fori_loop(0, 4, _m2, (0, 0))

                cv = mergev[pl.ds(jst * 16, 16)]
                ci = mergei[pl.ds(jst * 16, 16)]
                lane = _scal(plsc.all_reduce_ffs(cv == mx))
                widx = jnp.max(jnp.where(L == lane, ci, 0))
                plsc.store_scatter(wv, [_splat_i(r)], _splat_f(mx), mask=L == 0)
                plsc.store_scatter(wi, [_splat_i(r)], _splat_i(widx), mask=L == 0)
                cv2 = jnp.where(L == lane, jnp.float32(NEG), cv)
                mergev[pl.ds(jst * 16, 16)] = cv2
                plsc.store_scatter(pm, [_splat_i(jst)], _splat_f(jnp.max(cv2)),
                                   mask=L == 0)
                return 0
            lax.fori_loop(0, K, _peel, 0)

            @plsc.parallel_loop(0, 8, unroll=4)
            def _fin(i):
                x = wv[pl.ds(i * 16, 16)]
                outs_row[pl.ds(i * 16, 16)] = 1.0 / (1.0 + jnp.exp(-x))
                ix = wi[pl.ds(i * 16, 16)]
                q = ix // C
                outl_row[pl.ds(i * 16, 16)] = ix - q * C
                bidx[pl.ds(i * 16, 16)] = lax.shift_right_logical(q + img * N, 5)

            pltpu.sync_copy(sc.at[img], scalev)
            pltpu.async_copy(bx.at[bidx], gbox, semg).wait()
            scv = scalev[...]

            @plsc.parallel_loop(0, 32, unroll=4)
            def _box(j):
                f = j * 16 + L
                row = lax.shift_right_logical(f, 2)
                col = f & 3
                wl = plsc.load_gather(wi, [row])
                q = wl // C
                cb = (q & 31) * 4
                c0 = cb + col - jnp.where(col >= 2, 2, 0)
                g1 = plsc.load_gather(gbox, [row, c0])
                g2 = plsc.load_gather(gbox, [row, c0 + 2])
                sgn = jnp.where(col >= 2, 0.5, -0.5).astype(jnp.float32)
                outb_row[pl.ds(j * 16, 16)] = (g1 + sgn * g2) * scv

            pltpu.sync_copy(outs_row, out_s.at[img])
            pltpu.sync_copy(outl_row, out_l.at[img])
            pltpu.sync_copy(outb_row, out_b.at[img])
        return 0

    lax.fori_loop(0, 4, _image, 0)


def kernel(pred_logits, pred_boxes, target_sizes):
    flat = pred_logits.reshape(B * N * C)
    boxes2 = pred_boxes.reshape(B * N * 4 // 128, 128)
    img_h = target_sizes[:, 0].astype(jnp.float32)
    img_w = target_sizes[:, 1].astype(jnp.float32)
    scale = jnp.tile(jnp.stack([img_w, img_h, img_w, img_h], axis=1), (1, 4))

    mesh = plsc.VectorSubcoreMesh(core_axis_name="c", subcore_axis_name="s")
    out_s, out_l, out_b = pl.kernel(
        _body,
        out_type=(
            jax.ShapeDtypeStruct((B, 128), jnp.float32),
            jax.ShapeDtypeStruct((B, 128), jnp.int32),
            jax.ShapeDtypeStruct((B, 512), jnp.float32),
        ),
        mesh=mesh,
        compiler_params=pltpu.CompilerParams(needs_layout_passes=False),
        scratch_types=[
            pltpu.VMEM((WBUF,), jnp.float32),
            pltpu.VMEM((WBUF,), jnp.float32),
            pltpu.VMEM((32784,), jnp.int32),
            pltpu.VMEM((NBINS,), jnp.int32),
            pltpu.VMEM((NBINS,), jnp.int32),
            pltpu.VMEM((NBINS,), jnp.int32),
            pltpu.VMEM((CAP_T + 32,), jnp.float32),
            pltpu.VMEM((CAP_T + 32,), jnp.int32),
            pltpu.VMEM((16,), jnp.int32),
            pltpu.VMEM((16 * CAP_T,), jnp.float32),
            pltpu.VMEM((16 * CAP_T,), jnp.int32),
            pltpu.VMEM((256,), jnp.int32),
            pltpu.VMEM((CAP_M + 32,), jnp.float32),
            pltpu.VMEM((CAP_M + 32,), jnp.int32),
            pltpu.VMEM((64,), jnp.float32),
            pltpu.VMEM((128,), jnp.float32),
            pltpu.VMEM((128,), jnp.int32),
            pltpu.VMEM((128,), jnp.int32),
            pltpu.VMEM((128, 128), jnp.float32),
            pltpu.VMEM((16,), jnp.int32),
            pltpu.VMEM((128,), jnp.float32),
            pltpu.VMEM((128,), jnp.int32),
            pltpu.VMEM((512,), jnp.float32),
            pltpu.VMEM((16,), jnp.float32),
            pltpu.VMEM_SHARED((16 * NBINS,), jnp.int32),
            pltpu.VMEM_SHARED((2 * 16 * CAP_T,), jnp.float32),
            pltpu.VMEM_SHARED((2 * 16 * CAP_T,), jnp.int32),
            pltpu.VMEM_SHARED((512,), jnp.int32),
            pltpu.VMEM_SHARED((16,), jnp.int32),
            pltpu.SemaphoreType.DMA,
            pltpu.SemaphoreType.DMA,
            pltpu.SemaphoreType.DMA,
        ],
    )(flat, boxes2, scale)

    scores = out_s[:, :K]
    labels = out_l[:, :K]
    boxes = out_b.reshape(B, 128, 4)[:, :K, :]
    return scores, labels, boxes
